# baseline (speedup 1.0000x reference)
import functools

import jax
import jax.numpy as jnp
from jax import lax
from jax.experimental import pallas as pl
from jax.experimental.pallas import tpu as pltpu
from jax.experimental.pallas import tpu_sc as plsc

_LENGTH = 100000
_BATCH = 4096
_NBATCH = 25
_PADDED = _NBATCH * _BATCH
_D = 128
_PAD = _PADDED - _LENGTH

_NT = 16
_NPT = _PADDED // _NT
_NCHK = _NPT // 16
_R1BITS, _R2BITS = 12, 11
_R1 = 1 << _R1BITS
_R2 = 1 << _R2BITS
_CSH = 13
_CSH_MASK = (1 << _CSH) - 1
_NROW = _NPT // 128

def _iota16():
    return lax.iota(jnp.int32, 16)


def _runs(d_s):
    iota = _iota16()
    prv = d_s.at[jnp.maximum(iota - 1, 0)].get(mode="promise_in_bounds")
    nxt = d_s.at[jnp.minimum(iota + 1, 15)].get(mode="promise_in_bounds")
    is_first = jnp.logical_or(iota == 0, d_s != prv)
    is_last = jnp.logical_or(iota == 15, d_s != nxt)
    run_start = plsc.cummax(jnp.where(is_first, iota, 0))
    run_rank = iota - run_start
    return run_rank, is_last


def _zero_range(ref, n):
    def body(i, c):
        ref[pl.ds(i * 16, 16)] = jnp.broadcast_to(jnp.int32(0), (16,))
        return c

    lax.fori_loop(0, n // 16, body, 0)


def _radix_pass(src_load, store_j, hist_vm, off_vm, colrows, offstage, acc_vm,
                tot_vm, aux_vm, vs_vm, hists_sp, offmat_sp, totals_sp, sid,
                nbits, sem_h):
    radix = 1 << nbits
    sw = radix // 16
    _zero_range(hist_vm, radix)

    def a_one(i):
        digit, value = src_load(i)
        local = i * 16 + _iota16()
        comp = jnp.left_shift(digit, _CSH) | local
        comp_s, val_s = plsc.sort_key_val(comp, value)
        sl = pl.ds(i * 16, 16)
        vs_vm[sl] = val_s
        if store_j is not None:
            store_j(sl, comp_s)
        d_s = jnp.right_shift(comp_s, _CSH)
        run_rank, is_last = _runs(d_s)
        aux_vm[sl] = (jnp.left_shift(d_s, 5) | run_rank
                      | jnp.left_shift(is_last.astype(jnp.int32), 4))
        plsc.addupdate_scatter(hist_vm, [d_s], run_rank + 1, mask=is_last)

    def phase_a(i, c):
        a_one(4 * i)
        a_one(4 * i + 1)
        a_one(4 * i + 2)
        a_one(4 * i + 3)
        return c

    with jax.named_scope("phaseA"):
        lax.fori_loop(0, _NCHK // 4, phase_a, 0)

    pltpu.sync_copy(hist_vm.at[pl.ds(0, radix)],
                    hists_sp.at[sid, pl.ds(0, radix)])
    plsc.subcore_barrier()

    scope_b = jax.named_scope("phaseB")
    scope_b.__enter__()
    cps = [pltpu.async_copy(hists_sp.at[w, pl.ds(sid * sw, sw)],
                            colrows.at[w, pl.ds(0, sw)], sem_h)
           for w in range(_NT)]
    for cp in cps:
        cp.wait()
    _zero_range(acc_vm, sw)

    def prew(w, c):
        def add(i, c2):
            sl = pl.ds(i * 16, 16)
            offstage[w, sl] = acc_vm[sl]
            acc_vm[sl] = acc_vm[sl] + colrows[w, sl]
            return c2

        lax.fori_loop(0, sw // 16, add, 0)
        return c

    lax.fori_loop(0, _NT, prew, 0)

    def scan(i, carry):
        sl = pl.ds(i * 16, 16)
        chunk = acc_vm[sl]
        excl = (plsc.cumsum(chunk) - chunk) + carry

        def addrow(w, c2):
            offstage[w, sl] = offstage[w, sl] + excl
            return c2

        lax.fori_loop(0, _NT, addrow, 0)
        return carry + jnp.sum(chunk)

    stripe_total = lax.fori_loop(0, sw // 16, scan, jnp.int32(0))
    tot_vm[0, pl.ds(0, 16)] = jnp.broadcast_to(stripe_total, (16,))
    pltpu.sync_copy(tot_vm.at[0], totals_sp.at[sid])
    pltpu.sync_copy(offstage, offmat_sp.at[sid])
    plsc.subcore_barrier()

    pltpu.sync_copy(totals_sp, tot_vm)
    cps = [pltpu.async_copy(offmat_sp.at[t, sid, pl.ds(0, sw)],
                            colrows.at[t, pl.ds(0, sw)], sem_h)
           for t in range(_NT)]
    for cp in cps:
        cp.wait()

    def asm(t, carry):
        def add(i, c2):
            off_vm[pl.ds(t * sw + i * 16, 16)] = (
                colrows[t, pl.ds(i * 16, 16)] + carry)
            return c2

        lax.fori_loop(0, sw // 16, add, 0)
        return carry + tot_vm[t, pl.ds(0, 16)][0]

    lax.fori_loop(0, _NT, asm, jnp.int32(0))
    scope_b.__exit__(None, None, None)


def _phase_c(aux_vm, off_vm, pos2d):
    def c_one(i):
        aux = aux_vm[pl.ds(i * 16, 16)]
        d_s = jnp.right_shift(aux, 5)
        run_rank = aux & 15
        is_last = (jnp.right_shift(aux, 4) & 1) == 1
        offg = plsc.load_gather(off_vm, [d_s])
        pos = offg + run_rank
        plsc.addupdate_scatter(off_vm, [d_s], run_rank + 1, mask=is_last)
        pos2d[i // 8, pl.ds((i % 8) * 16, 16)] = pos

    def body(i, c):
        c_one(4 * i)
        c_one(4 * i + 1)
        c_one(4 * i + 2)
        c_one(4 * i + 3)
        return c

    with jax.named_scope("phaseC"):
        lax.fori_loop(0, _NCHK // 4, body, 0)


@functools.partial(
    pl.kernel,
    out_type=jax.ShapeDtypeStruct((_PADDED,), jnp.int32),
    mesh=plsc.VectorSubcoreMesh(core_axis_name="c", subcore_axis_name="s",
                                num_cores=1),
    compiler_params=pltpu.CompilerParams(needs_layout_passes=False),
    scratch_types=[
        pltpu.VMEM((_NPT,), jnp.float32),
        pltpu.VMEM((_NPT,), jnp.int32),
        pltpu.VMEM((_NPT,), jnp.int32),
        pltpu.VMEM((_NPT,), jnp.int32),
        pltpu.VMEM((_NROW, 128), jnp.int32),
        pltpu.VMEM((_R1,), jnp.int32),
        pltpu.VMEM((_R1,), jnp.int32),
        pltpu.VMEM((_NT, _R1 // 16), jnp.int32),
        pltpu.VMEM((_NT, _R1 // 16), jnp.int32),
        pltpu.VMEM((_R1 // 16,), jnp.int32),
        pltpu.VMEM((_NT, 16), jnp.int32),
        pltpu.VMEM_SHARED((_NT, _R1), jnp.int32),
        pltpu.VMEM_SHARED((_NT, _NT, _R1 // 16), jnp.int32),
        pltpu.VMEM_SHARED((_NT, 16), jnp.int32),
        pltpu.VMEM_SHARED((_PADDED,), jnp.int32),
        pltpu.VMEM_SHARED((_PADDED,), jnp.int32),
        pltpu.VMEM_SHARED((_PADDED,), jnp.int32),
        pltpu.SemaphoreType.DMA,
        pltpu.SemaphoreType.DMA,
        pltpu.SemaphoreType.DMA,
    ],
)
def _radix_argsort(u_hbm, order_hbm, uvm, aux_vm, ks_vm, js_vm, pos2d,
                   hist_vm, off_vm, colrows, offstage, acc_vm, tot_vm,
                   hists_sp, offmat_sp, totals_sp,
                   kbuf_sp, jbuf_sp, obuf_sp, sem_a, sem_b, sem_h):
    sid = lax.axis_index("s")
    base = sid * _NPT
    pltpu.sync_copy(u_hbm.at[pl.ds(base, _NPT)], uvm)

    def src1(i):
        u = uvm[pl.ds(i * 16, 16)]
        k = (u * 8388608.0).astype(jnp.int32)
        return k & (_R1 - 1), k

    def store_j1(sl, comp_s):
        js_vm[sl] = (comp_s & _CSH_MASK) + base

    _radix_pass(src1, store_j1, hist_vm, off_vm, colrows, offstage, acc_vm,
                tot_vm, aux_vm, ks_vm, hists_sp, offmat_sp, totals_sp, sid,
                _R1BITS, sem_h)
    _phase_c(aux_vm, off_vm, pos2d)

    with jax.named_scope("scat1"):
        for r0 in range(0, _NROW, 5):
            cps = []
            for r in range(r0, r0 + 5):
                cps.append(pltpu.async_copy(ks_vm.at[pl.ds(r * 128, 128)],
                                            kbuf_sp.at[pos2d.at[r]], sem_a))
                cps.append(pltpu.async_copy(js_vm.at[pl.ds(r * 128, 128)],
                                            jbuf_sp.at[pos2d.at[r]], sem_b))
            for cp in cps:
                cp.wait()
        plsc.subcore_barrier()

    cp_k = pltpu.async_copy(kbuf_sp.at[pl.ds(base, _NPT)], ks_vm, sem_a)
    cp_j = pltpu.async_copy(jbuf_sp.at[pl.ds(base, _NPT)], js_vm, sem_b)
    cp_k.wait()
    cp_j.wait()

    def src2(i):
        k = ks_vm[pl.ds(i * 16, 16)]
        j = js_vm[pl.ds(i * 16, 16)]
        return jnp.right_shift(k, _R1BITS), j

    _radix_pass(src2, None, hist_vm, off_vm, colrows, offstage, acc_vm,
                tot_vm, aux_vm, js_vm, hists_sp, offmat_sp, totals_sp, sid,
                _R2BITS, sem_h)
    _phase_c(aux_vm, off_vm, pos2d)

    with jax.named_scope("scat2"):
        for r0 in range(0, _NROW, 10):
            cps = [pltpu.async_copy(js_vm.at[pl.ds(r * 128, 128)],
                                    obuf_sp.at[pos2d.at[r]], sem_a)
                   for r in range(r0, r0 + 10)]
            for cp in cps:
                cp.wait()
        plsc.subcore_barrier()

    pltpu.sync_copy(obuf_sp.at[pl.ds(base, _NPT)],
                    order_hbm.at[pl.ds(base, _NPT)])


_NC, _NS = 2, 16
_NW = _NC * _NS
_RPW = _PADDED // _NW
_CH = 320
_NCH = _RPW // _CH
_NBUF = 3
_ZPW = 80
_ZW = _PAD // _ZPW


@functools.partial(
    pl.kernel,
    out_type=jax.ShapeDtypeStruct((_PADDED, _D), jnp.float32),
    mesh=plsc.VectorSubcoreMesh(core_axis_name="c", subcore_axis_name="s"),
    scratch_types=[
        pltpu.VMEM((_RPW,), jnp.int32),
        pltpu.VMEM((_CH, _D), jnp.float32),
        pltpu.VMEM((_CH, _D), jnp.float32),
        pltpu.VMEM((_CH, _D), jnp.float32),
        pltpu.SemaphoreType.DMA,
        pltpu.SemaphoreType.DMA,
        pltpu.SemaphoreType.DMA,
        pltpu.SemaphoreType.DMA,
        pltpu.SemaphoreType.DMA,
        pltpu.SemaphoreType.DMA,
    ],
)
def _gather_rows(table_hbm, idx_hbm, zeros_hbm, out_hbm,
                 idx_v, rows_a, rows_b, rows_c,
                 gsem_a, gsem_b, gsem_c, wsem_a, wsem_b, wsem_c):
    wid = lax.axis_index("s") * _NC + lax.axis_index("c")
    base = wid * _RPW
    row_bufs = (rows_a, rows_b, rows_c)
    gsems = (gsem_a, gsem_b, gsem_c)
    wsems = (wsem_a, wsem_b, wsem_c)

    pltpu.sync_copy(idx_hbm.at[pl.ds(base, _RPW)], idx_v)

    def clamp(i, c):
        v = idx_v[pl.ds(i * 16, 16)]
        idx_v[pl.ds(i * 16, 16)] = jnp.where(v >= _LENGTH, v - _LENGTH, v)
        return c

    lax.fori_loop(0, _RPW // 16, clamp, 0)

    def gat(g):
        b = g % _NBUF
        return pltpu.async_copy(table_hbm.at[idx_v.at[pl.ds(g * _CH, _CH)]],
                                row_bufs[b], gsems[b])

    gathers = [gat(0), gat(1)]
    writes = [None] * _NBUF
    for g in range(_NCH):
        nxt = g + _NBUF - 1
        if nxt < _NCH:
            b = nxt % _NBUF
            if writes[b] is not None:
                writes[b].wait()
                writes[b] = None
            gathers.append(gat(nxt))
        gathers[g].wait()
        writes[g % _NBUF] = pltpu.async_copy(
            row_bufs[g % _NBUF], out_hbm.at[pl.ds(base + g * _CH, _CH)],
            wsems[g % _NBUF])
    for w in writes:
        if w is not None:
            w.wait()

    @pl.when(wid < _ZW)
    def _():
        pltpu.sync_copy(zeros_hbm.at[pl.ds(wid * _ZPW, _ZPW)],
                        out_hbm.at[pl.ds(_LENGTH + wid * _ZPW, _ZPW)])


def kernel(epoch, table):
    key = jax.random.fold_in(jax.random.key(42), epoch)
    u = jax.random.uniform(key, (_LENGTH,))
    u_pad = jnp.concatenate(
        [u, jnp.full((_PAD,), (2.0**23 - 1) / 2.0**23, jnp.float32)])
    order = _radix_argsort(u_pad)
    zeros = jnp.zeros((_PAD, _D), jnp.float32)
    out = _gather_rows(table, order, zeros)
    xs = out.reshape(_NBATCH, _BATCH, _D)
    pad_mask = (jnp.arange(_PADDED, dtype=jnp.int32) >= _LENGTH).reshape(
        _NBATCH, _BATCH)
    return xs, pad_mask


# speedup vs baseline: 1.0267x; 1.0267x over previous
import functools

import jax
import jax.numpy as jnp
from jax import lax
from jax.experimental import pallas as pl
from jax.experimental.pallas import tpu as pltpu
from jax.experimental.pallas import tpu_sc as plsc

_LENGTH = 100000
_BATCH = 4096
_NBATCH = 25
_PADDED = _NBATCH * _BATCH
_D = 128
_PAD = _PADDED - _LENGTH

_NT = 16
_NPT = _PADDED // _NT
_NCHK = _NPT // 16
_R1BITS, _R2BITS = 12, 11
_R1 = 1 << _R1BITS
_R2 = 1 << _R2BITS
_CSH = 13
_CSH_MASK = (1 << _CSH) - 1
_NROW = _NPT // 128

def _iota16():
    return lax.iota(jnp.int32, 16)


def _zero_range(ref, n):
    def body(i, c):
        ref[pl.ds(i * 16, 16)] = jnp.broadcast_to(jnp.int32(0), (16,))
        return c

    lax.fori_loop(0, n // 16, body, 0)


def _radix_pass(src_load, hist_vm, off_vm, colrows, offstage, acc_vm,
                tot_vm, hists_sp, offmat_sp, totals_sp, sid, nbits, sem_h):
    radix = 1 << nbits
    sw = radix // 16
    _zero_range(hist_vm, radix)

    def a_one(i):
        digit = src_load(i)
        cnt, last = plsc.scan_count(digit)
        plsc.addupdate_scatter(hist_vm, [digit], cnt, mask=last)

    def phase_a(i, c):
        a_one(4 * i)
        a_one(4 * i + 1)
        a_one(4 * i + 2)
        a_one(4 * i + 3)
        return c

    with jax.named_scope("phaseA"):
        lax.fori_loop(0, _NCHK // 4, phase_a, 0)

    pltpu.sync_copy(hist_vm.at[pl.ds(0, radix)],
                    hists_sp.at[sid, pl.ds(0, radix)])
    plsc.subcore_barrier()

    scope_b = jax.named_scope("phaseB")
    scope_b.__enter__()
    cps = [pltpu.async_copy(hists_sp.at[w, pl.ds(sid * sw, sw)],
                            colrows.at[w, pl.ds(0, sw)], sem_h)
           for w in range(_NT)]
    for cp in cps:
        cp.wait()
    _zero_range(acc_vm, sw)

    def prew(w, c):
        def add(i, c2):
            sl = pl.ds(i * 16, 16)
            offstage[w, sl] = acc_vm[sl]
            acc_vm[sl] = acc_vm[sl] + colrows[w, sl]
            return c2

        lax.fori_loop(0, sw // 16, add, 0)
        return c

    lax.fori_loop(0, _NT, prew, 0)

    def scan(i, carry):
        sl = pl.ds(i * 16, 16)
        chunk = acc_vm[sl]
        excl = (plsc.cumsum(chunk) - chunk) + carry

        def addrow(w, c2):
            offstage[w, sl] = offstage[w, sl] + excl
            return c2

        lax.fori_loop(0, _NT, addrow, 0)
        return carry + jnp.sum(chunk)

    stripe_total = lax.fori_loop(0, sw // 16, scan, jnp.int32(0))
    tot_vm[0, pl.ds(0, 16)] = jnp.broadcast_to(stripe_total, (16,))
    pltpu.sync_copy(tot_vm.at[0], totals_sp.at[sid])
    pltpu.sync_copy(offstage, offmat_sp.at[sid])
    plsc.subcore_barrier()

    pltpu.sync_copy(totals_sp, tot_vm)
    cps = [pltpu.async_copy(offmat_sp.at[t, sid, pl.ds(0, sw)],
                            colrows.at[t, pl.ds(0, sw)], sem_h)
           for t in range(_NT)]
    for cp in cps:
        cp.wait()

    def asm(t, carry):
        def add(i, c2):
            off_vm[pl.ds(t * sw + i * 16, 16)] = (
                colrows[t, pl.ds(i * 16, 16)] + carry)
            return c2

        lax.fori_loop(0, sw // 16, add, 0)
        return carry + tot_vm[t, pl.ds(0, 16)][0]

    lax.fori_loop(0, _NT, asm, jnp.int32(0))
    scope_b.__exit__(None, None, None)


def _phase_c(src_load, off_vm, pos2d):
    def c_one(i):
        digit = src_load(i)
        cnt, last = plsc.scan_count(digit)
        offg = plsc.load_gather(off_vm, [digit])
        pos = offg + (cnt - 1)
        plsc.addupdate_scatter(off_vm, [digit], cnt, mask=last)
        pos2d[i // 8, pl.ds((i % 8) * 16, 16)] = pos

    def body(i, c):
        c_one(4 * i)
        c_one(4 * i + 1)
        c_one(4 * i + 2)
        c_one(4 * i + 3)
        return c

    with jax.named_scope("phaseC"):
        lax.fori_loop(0, _NCHK // 4, body, 0)


@functools.partial(
    pl.kernel,
    out_type=jax.ShapeDtypeStruct((_PADDED,), jnp.int32),
    mesh=plsc.VectorSubcoreMesh(core_axis_name="c", subcore_axis_name="s",
                                num_cores=1),
    compiler_params=pltpu.CompilerParams(needs_layout_passes=False),
    scratch_types=[
        pltpu.VMEM((_NPT,), jnp.float32),
        pltpu.VMEM((_NPT,), jnp.int32),
        pltpu.VMEM((_NPT,), jnp.int32),
        pltpu.VMEM((_NROW, 128), jnp.int32),
        pltpu.VMEM((_R1,), jnp.int32),
        pltpu.VMEM((_R1,), jnp.int32),
        pltpu.VMEM((_NT, _R1 // 16), jnp.int32),
        pltpu.VMEM((_NT, _R1 // 16), jnp.int32),
        pltpu.VMEM((_R1 // 16,), jnp.int32),
        pltpu.VMEM((_NT, 16), jnp.int32),
        pltpu.VMEM_SHARED((_NT, _R1), jnp.int32),
        pltpu.VMEM_SHARED((_NT, _NT, _R1 // 16), jnp.int32),
        pltpu.VMEM_SHARED((_NT, 16), jnp.int32),
        pltpu.VMEM_SHARED((_PADDED,), jnp.int32),
        pltpu.VMEM_SHARED((_PADDED,), jnp.int32),
        pltpu.VMEM_SHARED((_PADDED,), jnp.int32),
        pltpu.SemaphoreType.DMA,
        pltpu.SemaphoreType.DMA,
        pltpu.SemaphoreType.DMA,
    ],
)
def _radix_argsort(u_hbm, order_hbm, uvm, ks_vm, js_vm, pos2d,
                   hist_vm, off_vm, colrows, offstage, acc_vm, tot_vm,
                   hists_sp, offmat_sp, totals_sp,
                   kbuf_sp, jbuf_sp, obuf_sp, sem_a, sem_b, sem_h):
    sid = lax.axis_index("s")
    base = sid * _NPT
    pltpu.sync_copy(u_hbm.at[pl.ds(base, _NPT)], uvm)

    def src1_store(i):
        sl = pl.ds(i * 16, 16)
        u = uvm[sl]
        k = (u * 8388608.0).astype(jnp.int32)
        ks_vm[sl] = k
        js_vm[sl] = base + i * 16 + _iota16()
        return k & (_R1 - 1)

    def src1(i):
        return ks_vm[pl.ds(i * 16, 16)] & (_R1 - 1)

    _radix_pass(src1_store, hist_vm, off_vm, colrows, offstage, acc_vm,
                tot_vm, hists_sp, offmat_sp, totals_sp, sid, _R1BITS, sem_h)
    _phase_c(src1, off_vm, pos2d)

    with jax.named_scope("scat1"):
        for r0 in range(0, _NROW, 5):
            cps = []
            for r in range(r0, r0 + 5):
                cps.append(pltpu.async_copy(ks_vm.at[pl.ds(r * 128, 128)],
                                            kbuf_sp.at[pos2d.at[r]], sem_a))
                cps.append(pltpu.async_copy(js_vm.at[pl.ds(r * 128, 128)],
                                            jbuf_sp.at[pos2d.at[r]], sem_b))
            for cp in cps:
                cp.wait()
        plsc.subcore_barrier()

    cp_k = pltpu.async_copy(kbuf_sp.at[pl.ds(base, _NPT)], ks_vm, sem_a)
    cp_j = pltpu.async_copy(jbuf_sp.at[pl.ds(base, _NPT)], js_vm, sem_b)
    cp_k.wait()
    cp_j.wait()

    def src2(i):
        return jnp.right_shift(ks_vm[pl.ds(i * 16, 16)], _R1BITS)

    _radix_pass(src2, hist_vm, off_vm, colrows, offstage, acc_vm,
                tot_vm, hists_sp, offmat_sp, totals_sp, sid, _R2BITS, sem_h)
    _phase_c(src2, off_vm, pos2d)

    with jax.named_scope("scat2"):
        for r0 in range(0, _NROW, 10):
            cps = [pltpu.async_copy(js_vm.at[pl.ds(r * 128, 128)],
                                    obuf_sp.at[pos2d.at[r]], sem_a)
                   for r in range(r0, r0 + 10)]
            for cp in cps:
                cp.wait()
        plsc.subcore_barrier()

    pltpu.sync_copy(obuf_sp.at[pl.ds(base, _NPT)],
                    order_hbm.at[pl.ds(base, _NPT)])


_NC, _NS = 2, 16
_NW = _NC * _NS
_RPW = _PADDED // _NW
_CH = 320
_NCH = _RPW // _CH
_NBUF = 3
_ZPW = 80
_ZW = _PAD // _ZPW


@functools.partial(
    pl.kernel,
    out_type=jax.ShapeDtypeStruct((_PADDED, _D), jnp.float32),
    mesh=plsc.VectorSubcoreMesh(core_axis_name="c", subcore_axis_name="s"),
    scratch_types=[
        pltpu.VMEM((_RPW,), jnp.int32),
        pltpu.VMEM((_CH, _D), jnp.float32),
        pltpu.VMEM((_CH, _D), jnp.float32),
        pltpu.VMEM((_CH, _D), jnp.float32),
        pltpu.SemaphoreType.DMA,
        pltpu.SemaphoreType.DMA,
        pltpu.SemaphoreType.DMA,
        pltpu.SemaphoreType.DMA,
        pltpu.SemaphoreType.DMA,
        pltpu.SemaphoreType.DMA,
    ],
)
def _gather_rows(table_hbm, idx_hbm, zeros_hbm, out_hbm,
                 idx_v, rows_a, rows_b, rows_c,
                 gsem_a, gsem_b, gsem_c, wsem_a, wsem_b, wsem_c):
    wid = lax.axis_index("s") * _NC + lax.axis_index("c")
    base = wid * _RPW
    row_bufs = (rows_a, rows_b, rows_c)
    gsems = (gsem_a, gsem_b, gsem_c)
    wsems = (wsem_a, wsem_b, wsem_c)

    pltpu.sync_copy(idx_hbm.at[pl.ds(base, _RPW)], idx_v)

    def clamp(i, c):
        v = idx_v[pl.ds(i * 16, 16)]
        idx_v[pl.ds(i * 16, 16)] = jnp.where(v >= _LENGTH, v - _LENGTH, v)
        return c

    lax.fori_loop(0, _RPW // 16, clamp, 0)

    def gat(g):
        b = g % _NBUF
        return pltpu.async_copy(table_hbm.at[idx_v.at[pl.ds(g * _CH, _CH)]],
                                row_bufs[b], gsems[b])

    gathers = [gat(0), gat(1)]
    writes = [None] * _NBUF
    for g in range(_NCH):
        nxt = g + _NBUF - 1
        if nxt < _NCH:
            b = nxt % _NBUF
            if writes[b] is not None:
                writes[b].wait()
                writes[b] = None
            gathers.append(gat(nxt))
        gathers[g].wait()
        writes[g % _NBUF] = pltpu.async_copy(
            row_bufs[g % _NBUF], out_hbm.at[pl.ds(base + g * _CH, _CH)],
            wsems[g % _NBUF])
    for w in writes:
        if w is not None:
            w.wait()

    @pl.when(wid < _ZW)
    def _():
        pltpu.sync_copy(zeros_hbm.at[pl.ds(wid * _ZPW, _ZPW)],
                        out_hbm.at[pl.ds(_LENGTH + wid * _ZPW, _ZPW)])


def kernel(epoch, table):
    key = jax.random.fold_in(jax.random.key(42), epoch)
    u = jax.random.uniform(key, (_LENGTH,))
    u_pad = jnp.concatenate(
        [u, jnp.full((_PAD,), (2.0**23 - 1) / 2.0**23, jnp.float32)])
    order = _radix_argsort(u_pad)
    zeros = jnp.zeros((_PAD, _D), jnp.float32)
    out = _gather_rows(table, order, zeros)
    xs = out.reshape(_NBATCH, _BATCH, _D)
    pad_mask = (jnp.arange(_PADDED, dtype=jnp.int32) >= _LENGTH).reshape(
        _NBATCH, _BATCH)
    return xs, pad_mask
